# baseline (speedup 1.0000x reference)
import functools

import jax
import jax.numpy as jnp
from jax import lax
from jax.experimental import pallas as pl
from jax.experimental.pallas import tpu as pltpu
from jax.experimental.pallas import tpu_sc as plsc

DIM = 1024
N_EXPERTS = 64
TOP_K = 2
INTER = 256
T = 256
A = 512
SLOTS = 1024
CH = 8
EPS = 4


def _nt(a, b):
    return lax.dot_general(a, b, (((1,), (1,)), ((), ())),
                           preferred_element_type=jnp.float32)


def _route_body(x_ref, gw_ref, bias_ref, sw1_ref, sw2_ref, sw3_ref,
                xs_ref, cwm_ref, offs_ref, pcnt_ref, sh_ref):
    xv = x_ref[...]
    scores = _nt(xv, gw_ref[...]) + bias_ref[...]
    smax = jnp.max(scores, axis=1, keepdims=True)
    ex = jnp.exp(scores - smax)
    probs = ex / jnp.sum(ex, axis=1, keepdims=True)
    idxe = lax.broadcasted_iota(jnp.int32, (T, N_EXPERTS), 1)
    big = jnp.int32(10_000)
    m1 = jnp.max(probs, axis=1, keepdims=True)
    i1 = jnp.min(jnp.where(probs >= m1, idxe, big), axis=1, keepdims=True)
    pm = jnp.where(idxe == i1, jnp.float32(-1.0), probs)
    m2 = jnp.max(pm, axis=1, keepdims=True)
    i2 = jnp.min(jnp.where(pm >= m2, idxe, big), axis=1, keepdims=True)
    wsum = m1 + m2 + jnp.float32(1e-8)
    wn1 = m1 / wsum
    wn2 = m2 / wsum

    e_col = jnp.concatenate([i1, i2], axis=0)
    w_col = jnp.concatenate([wn1, wn2], axis=0)
    iota_e = lax.broadcasted_iota(jnp.int32, (1, N_EXPERTS), 1)
    amat = (e_col == iota_e).astype(jnp.float32)
    ra = lax.broadcasted_iota(jnp.int32, (A, A), 0)
    ca = lax.broadcasted_iota(jnp.int32, (A, A), 1)
    ltri = (ca <= ra).astype(jnp.float32)
    cum = jnp.dot(ltri, amat, preferred_element_type=jnp.float32)
    rank = jnp.sum(cum * amat, axis=1, keepdims=True) - 1.0
    counts = jnp.sum(amat, axis=0, keepdims=True)
    pcnt = jnp.floor((counts + 7.0) * 0.125) * 8.0
    re = lax.broadcasted_iota(jnp.int32, (N_EXPERTS, N_EXPERTS), 0)
    ce = lax.broadcasted_iota(jnp.int32, (N_EXPERTS, N_EXPERTS), 1)
    umat = (re < ce).astype(jnp.float32)
    offs = jnp.dot(pcnt, umat, preferred_element_type=jnp.float32)
    dest = jnp.sum(amat * offs, axis=1, keepdims=True) + rank
    dest_i = dest.astype(jnp.int32)
    iota_p = lax.broadcasted_iota(jnp.int32, (1, SLOTS), 1)
    omat = (dest_i == iota_p).astype(jnp.float32)
    xx = jnp.concatenate([xv, xv], axis=0)
    xs_ref[...] = lax.dot_general(
        omat, xx, (((0,), (0,)), ((), ())),
        precision=lax.Precision.HIGHEST,
        preferred_element_type=jnp.float32)
    rt = lax.broadcasted_iota(jnp.int32, (T, A), 0)
    caa = lax.broadcasted_iota(jnp.int32, (T, A), 1)
    tmat = (lax.rem(caa, jnp.int32(T)) == rt).astype(jnp.float32)
    cwm = lax.dot_general(tmat, omat * w_col, (((1,), (0,)), ((), ())),
                          precision=lax.Precision.HIGHEST,
                          preferred_element_type=jnp.float32)

    cwm_ref[...] = cwm
    offs_ref[...] = offs.astype(jnp.int32)
    pcnt_ref[...] = pcnt.astype(jnp.int32)

    s1 = _nt(xv, sw1_ref[...])
    s3 = _nt(xv, sw3_ref[...])
    hs = s1 * (1.0 / (1.0 + jnp.exp(-s1))) * s3
    sh_ref[...] = _nt(hs, sw2_ref[...])


def _route(x_flat, gate_weight, bias_row, sw1, sw2, sw3):
    outs = (
        jax.ShapeDtypeStruct((SLOTS, DIM), jnp.float32),
        jax.ShapeDtypeStruct((T, SLOTS), jnp.float32),
        jax.ShapeDtypeStruct((1, N_EXPERTS), jnp.int32),
        jax.ShapeDtypeStruct((1, N_EXPERTS), jnp.int32),
        jax.ShapeDtypeStruct((T, DIM), jnp.float32),
    )
    return pl.pallas_call(_route_body, out_shape=outs)(
        x_flat, gate_weight, bias_row, sw1, sw2, sw3)


def _shared_body(x_ref, sw1_ref, sw2_ref, sw3_ref, o_ref):
    xv = x_ref[...]
    s1 = _nt(xv, sw1_ref[...])
    s3 = _nt(xv, sw3_ref[...])
    hs = s1 * (1.0 / (1.0 + jnp.exp(-s1))) * s3
    o_ref[...] = _nt(hs, sw2_ref[...])


def _shared(x_flat, sw1, sw2, sw3):
    return pl.pallas_call(
        _shared_body,
        out_shape=jax.ShapeDtypeStruct((T, DIM), jnp.float32),
    )(x_flat, sw1, sw2, sw3)


def _expert_body(offs_ref, pcnt_ref, x_ref, w1_ref, w3_ref, w2_ref,
                 cwm_ref, sh_ref, o_ref, os_scr):
    g = pl.program_id(0)

    @pl.when(g == 0)
    def _zero():
        os_scr[...] = jnp.zeros((SLOTS, DIM), jnp.float32)

    for i in range(EPS):
        e = g * EPS + i
        off = offs_ref[e]
        cnt = pcnt_ref[e]
        w1b = w1_ref[i]
        w3b = w3_ref[i]
        w2b = w2_ref[i]

        def chunk(j, carry, off=off, w1b=w1b, w3b=w3b, w2b=w2b):
            base = pl.multiple_of(off + j * CH, CH)
            xs = x_ref[pl.ds(base, CH), :]
            h1 = _nt(xs, w1b)
            h3 = _nt(xs, w3b)
            h = h1 * (1.0 / (1.0 + jnp.exp(-h1))) * h3
            os_scr[pl.ds(base, CH), :] = _nt(h, w2b)
            return carry

        lax.fori_loop(0, cnt // CH, chunk, 0)

    @pl.when(g == N_EXPERTS // EPS - 1)
    def _combine():
        o_ref[...] = (jnp.dot(cwm_ref[...], os_scr[...],
                              preferred_element_type=jnp.float32)
                      + sh_ref[...])


def _experts(offs, pcnt, x_sorted, w1, w3, w2, cwm, shared):
    grid_spec = pltpu.PrefetchScalarGridSpec(
        num_scalar_prefetch=2,
        grid=(N_EXPERTS // EPS,),
        in_specs=[
            pl.BlockSpec((SLOTS, DIM), lambda g, *_: (0, 0)),
            pl.BlockSpec((EPS, INTER, DIM), lambda g, *_: (g, 0, 0)),
            pl.BlockSpec((EPS, INTER, DIM), lambda g, *_: (g, 0, 0)),
            pl.BlockSpec((EPS, DIM, INTER), lambda g, *_: (g, 0, 0)),
            pl.BlockSpec((T, SLOTS), lambda g, *_: (0, 0)),
            pl.BlockSpec((T, DIM), lambda g, *_: (0, 0)),
        ],
        out_specs=pl.BlockSpec((T, DIM), lambda g, *_: (0, 0)),
        scratch_shapes=[pltpu.VMEM((SLOTS, DIM), jnp.float32)],
    )
    return pl.pallas_call(
        _expert_body,
        grid_spec=grid_spec,
        out_shape=jax.ShapeDtypeStruct((T, DIM), jnp.float32),
    )(offs, pcnt, x_sorted, w1, w3, w2, cwm, shared)


def _sc_dispatch(x_flat, tid_sorted):
    info = plsc.get_sparse_core_info()
    nw = info.num_cores * info.num_subcores
    b_per_w = SLOTS // nw
    mesh = plsc.VectorSubcoreMesh(core_axis_name="c", subcore_axis_name="s")

    @functools.partial(
        pl.kernel,
        out_type=jax.ShapeDtypeStruct((SLOTS, DIM), jnp.float32),
        mesh=mesh,
        scratch_types=[
            pltpu.VMEM((b_per_w,), jnp.int32),
            pltpu.VMEM((b_per_w, DIM), jnp.float32),
            pltpu.SemaphoreType.DMA,
        ],
    )
    def k(x_hbm, idx_hbm, out_hbm, idx_v, rows_v, sem):
        wid = lax.axis_index("s") * info.num_cores + lax.axis_index("c")
        base = wid * b_per_w
        pltpu.sync_copy(idx_hbm.at[pl.ds(base, b_per_w)], idx_v)
        pltpu.async_copy(x_hbm.at[idx_v], rows_v, sem).wait()
        pltpu.sync_copy(rows_v, out_hbm.at[pl.ds(base, b_per_w)])

    return k(x_flat, tid_sorted)


def kernel(x, gate_weight, adaptive_bias, w1, w2, w3, sw1, sw2, sw3):
    b, s, d = x.shape
    x_flat = x.reshape(-1, d)
    bias_row = adaptive_bias.reshape(1, N_EXPERTS)
    x_sorted, cwm, offs, pcnt, shared = _route(
        x_flat, gate_weight, bias_row, sw1, sw2, sw3)
    out = _experts(offs.reshape(N_EXPERTS), pcnt.reshape(N_EXPERTS),
                   x_sorted, w1, w3, w2, cwm, shared)
    return out.reshape(b, s, d)
